# baseline (speedup 1.0000x reference)
import numpy as np
import jax
import jax.numpy as jnp
from jax.experimental import pallas as pl

L = 16
T = 2 ** 19
FPL = 2
BASE = 16.0
SCALE = 1.3819
N_RAYS = 4096
PRIMES = (np.uint32(1), np.uint32(2654435761), np.uint32(805459861))


def _hash_encode(x, table):
    outs = []
    for l in range(L):
        res = float(np.floor(BASE * SCALE ** l))
        pos = x * res
        pos0f = jnp.floor(pos)
        frac = pos - pos0f
        pos0 = pos0f.astype(jnp.uint32)
        feat = jnp.zeros((x.shape[0], FPL), dtype=x.dtype)
        for c in range(8):
            offs = np.array([(c >> 0) & 1, (c >> 1) & 1, (c >> 2) & 1], dtype=np.uint32)
            corner = pos0 + jnp.asarray(offs)
            idx = (corner[:, 0] * PRIMES[0]) ^ (corner[:, 1] * PRIMES[1]) ^ (corner[:, 2] * PRIMES[2])
            idx = (idx % np.uint32(T)).astype(jnp.int32)
            wgt = jnp.prod(jnp.where(jnp.asarray(offs) == 1, frac, 1.0 - frac), axis=-1)
            feat = feat + wgt[:, None] * table[l][idx]
        outs.append(feat)
    return jnp.concatenate(outs, axis=-1)


def _sh_encode(d):
    x, y, z = d[:, 0], d[:, 1], d[:, 2]
    xx, yy, zz = x * x, y * y, z * z
    xy, yz, xz = x * y, y * z, x * z
    c = [
        0.28209479177387814 * jnp.ones_like(x),
        -0.48860251190291987 * y,
        0.48860251190291987 * z,
        -0.48860251190291987 * x,
        1.0925484305920792 * xy,
        -1.0925484305920792 * yz,
        0.94617469575755997 * zz - 0.31539156525252005,
        -1.0925484305920792 * xz,
        0.54627421529603959 * (xx - yy),
        0.59004358992664352 * y * (-3.0 * xx + yy),
        2.8906114426405538 * xy * z,
        0.45704579946446572 * y * (1.0 - 5.0 * zz),
        0.3731763325901154 * z * (5.0 * zz - 3.0),
        0.45704579946446572 * x * (1.0 - 5.0 * zz),
        1.4453057213202769 * z * (xx - yy),
        0.59004358992664352 * x * (-xx + 3.0 * yy),
    ]
    return jnp.stack(c, axis=-1)


def kernel(xyzs, dirs, deltas, table, w1, b1, w2, b2, wr1, br1, wr2, br2, wr3, br3, segment_ids):
    h = _hash_encode(xyzs, table)
    g = jax.nn.relu(h @ w1 + b1) @ w2 + b2
    sigma = jnp.exp(g[:, 0])
    geo = g[:, 1:]
    dn = dirs / (jnp.linalg.norm(dirs, axis=-1, keepdims=True) + 1e-8)
    de = _sh_encode(dn)
    ri = jnp.concatenate([de, geo], axis=-1)
    h2 = jax.nn.relu(ri @ wr1 + br1)
    h2 = jax.nn.relu(h2 @ wr2 + br2)
    rgb = jax.nn.sigmoid(h2 @ wr3 + br3)
    dt = deltas[:, 0] * 0.01
    ts = deltas[:, 1]
    s = sigma * dt
    cs = jnp.cumsum(s)
    excl = cs - s
    off = jax.ops.segment_min(excl, segment_ids, num_segments=N_RAYS)[segment_ids]
    trans = jnp.exp(-(excl - off))
    alpha = 1.0 - jnp.exp(-s)
    w = alpha * trans
    image = jax.ops.segment_sum(w[:, None] * rgb, segment_ids, num_segments=N_RAYS)
    depth = jax.ops.segment_sum(w * ts, segment_ids, num_segments=N_RAYS)
    return image, depth
